# Initial kernel scaffold; baseline (speedup 1.0000x reference)
import jax
import jax.numpy as jnp
from jax.experimental import pallas as pl


def kernel(geom_feats, x):
    raise NotImplementedError("write your pallas kernel here")


# speedup vs baseline: 2.0352x; 2.0352x over previous
import functools

import jax
import jax.numpy as jnp
import numpy as np
from jax import lax
from jax.experimental import pallas as pl
from jax.experimental.pallas import tpu as pltpu
from jax.experimental.pallas import tpu_sc as plsc

B = 2
C = 80
NPRIME = 506880
PPB = NPRIME // B
NS = 16
PPT = PPB // NS
CH = 16
NPASS = C // CH
GRID = 200
CELLS = GRID * GRID
DUMP = CELLS
ACC_ROWS = CELLS + 8
CHUNK = 96
NCHUNK = PPT // CHUNK
CPS = 11
SLAB = CHUNK * CPS
NSLAB = PPT // SLAB
ROWS_PER_TILE = CELLS // NS

_DX = np.float32(0.005)
_DZ = np.float32(1.0)

_mesh = plsc.VectorSubcoreMesh(core_axis_name="c", subcore_axis_name="s")


@functools.partial(
    pl.kernel,
    mesh=_mesh,
    compiler_params=pltpu.CompilerParams(use_tc_tiling_on_sc=False),
    out_type=jax.ShapeDtypeStruct((B, NPASS, CELLS, CH), jnp.float32),
    scratch_types=[
        pltpu.VMEM((3, PPT), jnp.float32),
        pltpu.VMEM((NCHUNK, CHUNK), jnp.int32),
        pltpu.VMEM((SLAB, CH), jnp.float32),
        pltpu.VMEM_SHARED((ACC_ROWS, CH), jnp.float32),
    ],
)
def _bevpool_sc(g_hbm, x_hbm, z_hbm, out_hbm, gbuf, idxbuf, xbuf, acc):
    c = lax.axis_index("c")
    s = lax.axis_index("s")
    gbase = c * PPB + s * PPT

    pltpu.sync_copy(g_hbm.at[:, pl.ds(gbase, PPT)], gbuf)

    def _voxelize(r, carry):
        for cc in range(CHUNK // 16):
            o = r * CHUNK + cc * 16
            vx = gbuf[0, pl.ds(o, 16)]
            vy = gbuf[1, pl.ds(o, 16)]
            vz = gbuf[2, pl.ds(o, 16)]
            ix = (vx / _DX).astype(jnp.int32)
            iy = (vy / _DX).astype(jnp.int32)
            iz = (vz / _DZ).astype(jnp.int32)
            kept = (
                (ix >= 0) & (ix < GRID)
                & (iy >= 0) & (iy < GRID)
                & (iz >= 0) & (iz < 1)
            )
            lin = ix * GRID + iy
            idxbuf[r, pl.ds(cc * 16, 16)] = jnp.where(kept, lin, DUMP)
        return carry

    lax.fori_loop(0, NCHUNK, _voxelize, 0)

    for p in range(NPASS):
        pltpu.sync_copy(z_hbm, acc.at[pl.ds(s * ROWS_PER_TILE, ROWS_PER_TILE)])
        plsc.subcore_barrier()

        def _slab(t, carry, p=p):
            pltpu.sync_copy(
                x_hbm.at[pl.ds(gbase + t * SLAB, SLAB), pl.ds(p * CH, CH)],
                xbuf,
            )
            for j in range(CPS):
                pltpu.sync_copy(
                    xbuf.at[pl.ds(j * CHUNK, CHUNK)],
                    acc.at[idxbuf.at[t * CPS + j]],
                    add=True,
                )
            return carry

        lax.fori_loop(0, NSLAB, _slab, 0)
        plsc.subcore_barrier()

        pltpu.sync_copy(
            acc.at[pl.ds(s * ROWS_PER_TILE, ROWS_PER_TILE)],
            out_hbm.at[c, p, pl.ds(s * ROWS_PER_TILE, ROWS_PER_TILE)],
        )
        plsc.subcore_barrier()


def kernel(geom_feats, x):
    n = NPRIME
    x2d = x.reshape(n, C)
    g = geom_feats.reshape(n, 3).T
    zeros = jnp.zeros((ROWS_PER_TILE, CH), jnp.float32)
    out = _bevpool_sc(g, x2d, zeros)
    return out.transpose(0, 1, 3, 2).reshape(B, C, GRID, GRID)
